# Initial kernel scaffold; baseline (speedup 1.0000x reference)
import jax
import jax.numpy as jnp
from jax.experimental import pallas as pl


def kernel(x, edge_index, pseudo, W, W_root, bias, lin_W, lin_b):
    raise NotImplementedError("write your pallas kernel here")


# speedup vs baseline: 10.3211x; 10.3211x over previous
import functools

import jax
import jax.numpy as jnp
from jax import lax
from jax.experimental import pallas as pl
from jax.experimental.pallas import tpu as pltpu
from jax.experimental.pallas import tpu_sc as plsc

N = 100000
E = 3200000
K = 5
OUT = 16

NC = 2
NS = 16
NW = NC * NS
EPT = E // NW
CHUNK = 800
NCHUNK = EPT // CHUNK
GROUPS = CHUNK // 16
SCAT_ROWS = 8
SCAT_C = CHUNK // SCAT_ROWS
ROWS_PT = 6256
NPAD = NS * ROWS_PT
ZBLK = 136


def _sc_body(src_hbm, dst_hbm, pseudo_hbm, x_hbm, w2_hbm,
             out_hbm, w2, srcbuf, dstbuf, pbuf, xchunk, msgbuf, zbuf, sem,
             xsh, agg):
  c = lax.axis_index("c")
  s = lax.axis_index("s")
  wid = c * NS + s

  pltpu.sync_copy(w2_hbm, w2)

  @pl.when(s == 0)
  def _():
    pltpu.sync_copy(x_hbm, xsh)

  def zrow(i, _):
    zbuf[i, :] = jnp.zeros((16,), jnp.float32)
    return 0
  lax.fori_loop(0, ZBLK, zrow, 0)
  rows0 = s * ROWS_PT
  def zcopy(k, _):
    pltpu.sync_copy(zbuf, agg.at[pl.ds(rows0 + k * ZBLK, ZBLK)])
    return 0
  lax.fori_loop(0, ROWS_PT // ZBLK, zcopy, 0)
  plsc.subcore_barrier()

  iota = lax.iota(jnp.int32, 16)
  ebase = wid * EPT

  def chunk_body(j, _):
    off = ebase + j * CHUNK
    row0 = wid * (EPT // SCAT_C) + j * SCAT_ROWS
    pltpu.sync_copy(src_hbm.at[pl.ds(row0, SCAT_ROWS)], srcbuf)
    pltpu.sync_copy(dst_hbm.at[pl.ds(row0, SCAT_ROWS)], dstbuf)
    pltpu.sync_copy(pseudo_hbm.at[pl.ds(off * 3, CHUNK * 3)], pbuf)
    handles = [pltpu.async_copy(xsh.at[srcbuf.at[r]], xchunk.at[r], sem)
               for r in range(SCAT_ROWS)]
    for h in handles:
      h.wait()

    def group(i, _):
      base = i * 16
      e = base + iota
      er = e // SCAT_C
      ec = e - er * SCAT_C
      x_v = plsc.load_gather(xchunk, [er, ec])
      p3 = base * 3
      p0 = plsc.load_gather(pbuf, [iota * 3 + p3]) * (K - 1.0)
      p1 = plsc.load_gather(pbuf, [iota * 3 + (p3 + 1)]) * (K - 1.0)
      p2 = plsc.load_gather(pbuf, [iota * 3 + (p3 + 2)]) * (K - 1.0)
      lo0 = jnp.minimum(p0.astype(jnp.int32), K - 2)
      lo1 = jnp.minimum(p1.astype(jnp.int32), K - 2)
      lo2 = jnp.minimum(p2.astype(jnp.int32), K - 2)
      f0 = p0 - lo0.astype(jnp.float32)
      f1 = p1 - lo1.astype(jnp.float32)
      f2 = p2 - lo2.astype(jnp.float32)
      g0 = 1.0 - f0
      g1 = 1.0 - f1
      g2 = 1.0 - f2
      cellw = (lo0 + 5 * lo1 + 25 * lo2) * 16
      msgs = [jnp.zeros((16,), jnp.float32) for _ in range(OUT)]
      for bits in range(8):
        dx, dy, dz = bits & 1, (bits >> 1) & 1, (bits >> 2) & 1
        b = ((f0 if dx else g0) * (f1 if dy else g1) * (f2 if dz else g2))
        bx = b * x_v
        widx = cellw + (dx + 5 * dy + 25 * dz) * 16
        for o in range(OUT):
          w = plsc.load_gather(w2, [widx + o])
          msgs[o] = msgs[o] + w * bx
      for o in range(OUT):
        plsc.store_scatter(msgbuf, [er, ec, jnp.full((16,), o, jnp.int32)],
                           msgs[o])
      return 0
    lax.fori_loop(0, GROUPS, group, 0)

    for r in range(SCAT_ROWS):
      pltpu.sync_copy(msgbuf.at[r], agg.at[dstbuf.at[r]], add=True)
    return 0
  lax.fori_loop(0, NCHUNK, chunk_body, 0)

  plsc.subcore_barrier()
  pltpu.sync_copy(agg.at[pl.ds(rows0, ROWS_PT)],
                  out_hbm.at[c].at[pl.ds(rows0, ROWS_PT)])


@jax.jit
def _sc_aggregate(src, dst2d, pseudo_flat, x_flat, w2_flat):
  mesh = plsc.VectorSubcoreMesh(core_axis_name="c", subcore_axis_name="s")
  f = pl.kernel(
      _sc_body,
      out_type=jax.ShapeDtypeStruct((NC, NPAD, OUT), jnp.float32),
      mesh=mesh,
      scratch_types=[
          pltpu.VMEM((K ** 3 * OUT,), jnp.float32),
          pltpu.VMEM((SCAT_ROWS, SCAT_C), jnp.int32),
          pltpu.VMEM((SCAT_ROWS, SCAT_C), jnp.int32),
          pltpu.VMEM((CHUNK * 3,), jnp.float32),
          pltpu.VMEM((SCAT_ROWS, SCAT_C), jnp.float32),
          pltpu.VMEM((SCAT_ROWS, SCAT_C, OUT), jnp.float32),
          pltpu.VMEM((ZBLK, OUT), jnp.float32),
          pltpu.SemaphoreType.DMA,
          pltpu.VMEM_SHARED((N,), jnp.float32),
          pltpu.VMEM_SHARED((NPAD, OUT), jnp.float32),
      ],
      compiler_params=pltpu.CompilerParams(needs_layout_passes=False,
                                           use_tc_tiling_on_sc=False),
  )
  return f(src, dst2d, pseudo_flat, x_flat, w2_flat)


def _head_body(p0_ref, p1_ref, x_ref, wr_ref, b_ref, lw_ref, lb_ref, o_ref):
  h = p0_ref[...] + p1_ref[...] + x_ref[...] * wr_ref[...] + b_ref[...]
  h = jnp.where(h > 0, h, jnp.exp(jnp.minimum(h, 0.0)) - 1.0)
  q = jnp.dot(h, lw_ref[...], preferred_element_type=jnp.float32) + lb_ref[...]
  sq = jnp.sum(q * q, axis=-1, keepdims=True)
  o_ref[...] = q / (jnp.sqrt(sq) + 1e-4)


@jax.jit
def _head(p0, p1, x, w_root, bias, lin_w, lin_b):
  blk = 2000
  grid = (N // blk,)
  return pl.pallas_call(
      _head_body,
      grid=grid,
      in_specs=[
          pl.BlockSpec((blk, OUT), lambda i: (i, 0)),
          pl.BlockSpec((blk, OUT), lambda i: (i, 0)),
          pl.BlockSpec((blk, 1), lambda i: (i, 0)),
          pl.BlockSpec((1, OUT), lambda i: (0, 0)),
          pl.BlockSpec((1, OUT), lambda i: (0, 0)),
          pl.BlockSpec((OUT, 4), lambda i: (0, 0)),
          pl.BlockSpec((1, 4), lambda i: (0, 0)),
      ],
      out_specs=pl.BlockSpec((blk, 4), lambda i: (i, 0)),
      out_shape=jax.ShapeDtypeStruct((N, 4), jnp.float32),
  )(p0, p1, x, w_root, bias, lin_w, lin_b)


def kernel(x, edge_index, pseudo, W, W_root, bias, lin_W, lin_b):
  src = edge_index[0].astype(jnp.int32).reshape(E // SCAT_C, SCAT_C)
  dst2d = edge_index[1].astype(jnp.int32).reshape(E // SCAT_C, SCAT_C)
  pseudo_flat = pseudo.reshape(-1)
  x_flat = x.reshape(-1)
  w2_flat = W.reshape(-1)
  partials = _sc_aggregate(src, dst2d, pseudo_flat, x_flat, w2_flat)
  out = _head(partials[0, :N], partials[1, :N], x,
              W_root.reshape(1, OUT), bias.reshape(1, OUT),
              lin_W, lin_b.reshape(1, 4))
  return out.reshape(N, 1, 4)
